# baseline (speedup 1.0000x reference)
import functools

import jax
import jax.numpy as jnp
import numpy as np
from jax import lax
from jax.experimental import pallas as pl
from jax.experimental.pallas import tpu as pltpu
from jax.experimental.pallas import tpu_sc as plsc

B, T, C_IN = 1024, 200, 131
C_OUT = 84
ROWS = B * T

_COLS = np.array(
    [i for i in range(3, 66) if (i - 3) % 3 != 2]
    + [i for i in range(68, 131) if (i - 68) % 3 != 2],
    dtype=np.int32,
)
assert _COLS.shape[0] == C_OUT

R = 128
_IDX_NP = (np.arange(R, dtype=np.int32)[:, None] * C_IN + _COLS[None, :]).reshape(-1)

NC = 2
NS = 16
NW = NC * NS
ROWS_PER_W = ROWS // NW
STEPS = ROWS_PER_W // R
IN_CHUNK = R * C_IN
OUT_CHUNK = R * C_OUT
N_VECS = OUT_CHUNK // 16


def _sc_reduce(x_flat, idx):
    mesh = plsc.VectorSubcoreMesh(core_axis_name="c", subcore_axis_name="s")

    @functools.partial(
        pl.kernel,
        mesh=mesh,
        out_type=jax.ShapeDtypeStruct((ROWS * C_OUT,), jnp.float32),
        scratch_types=[
            pltpu.VMEM((OUT_CHUNK,), jnp.int32),
            pltpu.VMEM((IN_CHUNK,), jnp.float32),
            pltpu.VMEM((OUT_CHUNK,), jnp.float32),
        ],
        compiler_params=pltpu.CompilerParams(needs_layout_passes=False),
    )
    def k(x_hbm, idx_hbm, out_hbm, idx_v, in_v, out_v):
        wid = lax.axis_index("s") * NC + lax.axis_index("c")
        in_base = wid * (ROWS_PER_W * C_IN)
        out_base = wid * (ROWS_PER_W * C_OUT)
        pltpu.sync_copy(idx_hbm, idx_v)

        def step(s, carry):
            pltpu.sync_copy(x_hbm.at[pl.ds(in_base + s * IN_CHUNK, IN_CHUNK)], in_v)

            def body(j, carry2):
                iv = idx_v[pl.ds(j * 16, 16)]
                out_v[pl.ds(j * 16, 16)] = plsc.load_gather(in_v, [iv])
                return carry2

            lax.fori_loop(0, N_VECS, body, 0, unroll=8)
            pltpu.sync_copy(out_v, out_hbm.at[pl.ds(out_base + s * OUT_CHUNK, OUT_CHUNK)])
            return carry

        lax.fori_loop(0, STEPS, step, 0)

    return k(x_flat, idx)


def kernel(X):
    x_flat = X.reshape(-1)
    idx = jnp.asarray(_IDX_NP)
    out_flat = _sc_reduce(x_flat, idx)
    return out_flat.reshape(B, T, C_OUT)


# speedup vs baseline: 1.3153x; 1.3153x over previous
import functools

import jax
import jax.numpy as jnp
import numpy as np
from jax import lax
from jax.experimental import pallas as pl
from jax.experimental.pallas import tpu as pltpu
from jax.experimental.pallas import tpu_sc as plsc

B, T, C_IN = 1024, 200, 131
C_OUT = 84
ROWS = B * T

_COLS = np.array(
    [i for i in range(3, 66) if (i - 3) % 3 != 2]
    + [i for i in range(68, 131) if (i - 68) % 3 != 2],
    dtype=np.int32,
)
assert _COLS.shape[0] == C_OUT

GROUP_OUT = 336
N_PAT = GROUP_OUT // 16
_IDX_NP = np.array(
    [(p // C_OUT) * C_IN + _COLS[p % C_OUT] for p in range(GROUP_OUT)],
    dtype=np.int32,
)

NC = 2
NS = 16
NW = NC * NS
ROWS_PER_W = ROWS // NW
R = 200
STEPS = ROWS_PER_W // R
PAIRS = STEPS // 2
GROUPS = R // 4
IN_CHUNK = R * C_IN
OUT_CHUNK = R * C_OUT


def _sc_reduce(x_flat, idx):
    mesh = plsc.VectorSubcoreMesh(core_axis_name="c", subcore_axis_name="s")

    @functools.partial(
        pl.kernel,
        mesh=mesh,
        out_type=jax.ShapeDtypeStruct((ROWS * C_OUT,), jnp.float32),
        scratch_types=[
            pltpu.VMEM((GROUP_OUT,), jnp.int32),
            pltpu.VMEM((IN_CHUNK,), jnp.float32),
            pltpu.VMEM((IN_CHUNK,), jnp.float32),
            pltpu.VMEM((OUT_CHUNK,), jnp.float32),
            pltpu.VMEM((OUT_CHUNK,), jnp.float32),
            pltpu.SemaphoreType.DMA,
            pltpu.SemaphoreType.DMA,
            pltpu.SemaphoreType.DMA,
            pltpu.SemaphoreType.DMA,
        ],
        compiler_params=pltpu.CompilerParams(needs_layout_passes=False),
    )
    def k(x_hbm, idx_hbm, out_hbm, idx_v, in0, in1, out0, out1,
          sin0, sin1, sout0, sout1):
        wid = lax.axis_index("s") * NC + lax.axis_index("c")
        in_base = wid * (ROWS_PER_W * C_IN)
        out_base = wid * (ROWS_PER_W * C_OUT)
        pltpu.sync_copy(idx_hbm, idx_v)
        pats = [idx_v[pl.ds(j * 16, 16)] for j in range(N_PAT)]

        def start_in(s, buf, sem):
            s = jnp.minimum(s, STEPS - 1)
            pltpu.async_copy(
                x_hbm.at[pl.ds(in_base + s * IN_CHUNK, IN_CHUNK)], buf, sem)

        def wait_in(buf, sem):
            pltpu.make_async_copy(
                x_hbm.at[pl.ds(0, IN_CHUNK)], buf, sem).wait()

        def start_out(buf, s, sem):
            pltpu.async_copy(
                buf, out_hbm.at[pl.ds(out_base + s * OUT_CHUNK, OUT_CHUNK)],
                sem)

        def wait_out(buf, sem):
            pltpu.make_async_copy(
                buf, out_hbm.at[pl.ds(0, OUT_CHUNK)], sem).wait()

        def compute(in_ref, out_ref):
            def grp(g, c):
                base = jnp.full((16,), g * (4 * C_IN), jnp.int32)
                for j in range(N_PAT):
                    out_ref[pl.ds(g * GROUP_OUT + j * 16, 16)] = (
                        plsc.load_gather(in_ref, [pats[j] + base]))
                return c
            lax.fori_loop(0, GROUPS, grp, 0)

        start_in(0, in0, sin0)
        start_in(1, in1, sin1)
        wait_in(in0, sin0)
        compute(in0, out0)
        start_out(out0, 0, sout0)
        start_in(2, in0, sin0)
        wait_in(in1, sin1)
        compute(in1, out1)
        start_out(out1, 1, sout1)
        start_in(3, in1, sin1)

        def pair(t, c):
            s0 = 2 * t
            wait_in(in0, sin0)
            wait_out(out0, sout0)
            compute(in0, out0)
            start_out(out0, s0, sout0)
            start_in(s0 + 2, in0, sin0)
            wait_in(in1, sin1)
            wait_out(out1, sout1)
            compute(in1, out1)
            start_out(out1, s0 + 1, sout1)
            start_in(s0 + 3, in1, sin1)
            return c

        lax.fori_loop(1, PAIRS, pair, 0)

        wait_in(in0, sin0)
        wait_in(in1, sin1)
        wait_out(out0, sout0)
        wait_out(out1, sout1)

    return k(x_flat, idx)


def kernel(X):
    x_flat = X.reshape(-1)
    idx = jnp.asarray(_IDX_NP)
    out_flat = _sc_reduce(x_flat, idx)
    return out_flat.reshape(B, T, C_OUT)
